# baseline (speedup 1.0000x reference)
import jax
import jax.numpy as jnp
from jax.experimental import pallas as pl


def _copy_block(x_ref, o_ref):
    o_ref[...] = x_ref[...]


def kernel(x_dict, edge_index, entity_emb, rel_emb):
    del x_dict, edge_index
    n_ent, d_ent = entity_emb.shape
    blk = 8000
    ent = pl.pallas_call(
        _copy_block,
        grid=(n_ent // blk,),
        in_specs=[pl.BlockSpec((blk, d_ent), lambda i: (i, 0))],
        out_specs=pl.BlockSpec((blk, d_ent), lambda i: (i, 0)),
        out_shape=jax.ShapeDtypeStruct((n_ent, d_ent), entity_emb.dtype),
    )(entity_emb)
    rel = pl.pallas_call(
        _copy_block,
        out_shape=jax.ShapeDtypeStruct(rel_emb.shape, rel_emb.dtype),
    )(rel_emb)
    return (ent, rel)
